# baseline (speedup 1.0000x reference)
import functools

import jax
import jax.numpy as jnp
from jax import lax
from jax.experimental import pallas as pl
from jax.experimental.pallas import tpu as pltpu
from jax.experimental.pallas import tpu_sc as plsc

D = 64
_SCALE = 8.0
_NC, _NS, _L = 2, 16, 16
V = 1_000_000
_VMAIN = 999_936

_W1A = 4096
_W1B = 256


@jax.jit
def _sc_rowify(tf2d, tail):
    mesh = plsc.VectorSubcoreMesh(core_axis_name="c", subcore_axis_name="s")

    @functools.partial(
        pl.kernel,
        mesh=mesh,
        out_type=jax.ShapeDtypeStruct((V, D), jnp.float32),
        scratch_types=(
            [pltpu.VMEM((D, _W1B), jnp.float32) for _ in range(2)]
            + [pltpu.VMEM((_W1B, D), jnp.float32) for _ in range(2)]
            + [pltpu.VMEM((D, D), jnp.float32)]
            + [pltpu.SemaphoreType.DMA for _ in range(4)]
        ),
        compiler_params=pltpu.CompilerParams(
            use_tc_tiling_on_sc=False, needs_layout_passes=False
        ),
    )
    def k(tf_hbm, tail_hbm, tr_hbm, in0, in1, out0, out1, tv, gi0, gi1, go0, go1):
        wid = lax.axis_index("s") * _NC + lax.axis_index("c")
        iotas = [lax.iota(jnp.int32, _L) + (dq * _L) for dq in range(4)]
        ins, outs = (in0, in1), (out0, out1)
        gis, gos = (gi0, gi1), (go0, go1)

        def fire_in(t, b):
            pltpu.async_copy(
                tf_hbm.at[:, pl.ds(t * _W1B, _W1B)], ins[b], gis[b]
            )

        def wait_in(t, b):
            pltpu.make_async_copy(
                tf_hbm.at[:, pl.ds(t * _W1B, _W1B)], ins[b], gis[b]
            ).wait()

        def fire_out(t, b):
            pltpu.async_copy(
                outs[b], tr_hbm.at[pl.ds(t * _W1B, _W1B), :], gos[b]
            )

        def wait_out(t, b):
            pltpu.make_async_copy(
                outs[b], tr_hbm.at[pl.ds(t * _W1B, _W1B), :], gos[b]
            ).wait()

        def transpose(b):
            src, dst = ins[b], outs[b]

            @plsc.parallel_loop(0, _W1B, unroll=2)
            def _(r):
                col = jnp.full((_L,), 1, jnp.int32) * r
                for dq in range(4):
                    dst[r, pl.ds(dq * _L, _L)] = plsc.load_gather(
                        src, [iotas[dq], col]
                    )

        fire_in(wid, 0)
        fire_in(wid + 32, 1)

        def pair_body(k2, carry):
            for b in range(2):
                kk = k2 * 2 + b
                t = wid + 32 * kk

                @pl.when(kk >= 2)
                def _():
                    wait_out(t - 64, b)

                wait_in(t, b)
                transpose(b)
                fire_out(t, b)

                @pl.when(kk + 2 < 122)
                def _():
                    fire_in(t + 64, b)
            return carry

        lax.fori_loop(0, 61, pair_body, 0)
        wait_out(wid + 32 * 120, 0)
        wait_out(wid + 32 * 121, 1)

        @pl.when(wid < 2)
        def _():
            t = 3904 + wid
            pltpu.sync_copy(tf_hbm.at[:, pl.ds(t * _W1B, _W1B)], in0)
            transpose(0)
            pltpu.sync_copy(out0, tr_hbm.at[pl.ds(t * _W1B, _W1B), :])

        @pl.when(wid == 2)
        def _():
            pltpu.sync_copy(tail_hbm, tv)
            pltpu.sync_copy(tv, tr_hbm.at[pl.ds(_VMAIN, D), :])

    return k(tf2d, tail)


@jax.jit
def _sc_lookup(x2, t_row):
    S, B = x2.shape
    mesh = plsc.VectorSubcoreMesh(core_axis_name="c", subcore_axis_name="s")

    @functools.partial(
        pl.kernel,
        mesh=mesh,
        out_type=jax.ShapeDtypeStruct((S * D * B,), jnp.float32),
        scratch_types=(
            [pltpu.VMEM((S, 128), jnp.int32)]
            + [pltpu.VMEM((128, D), jnp.float32) for _ in range(2)]
            + [pltpu.VMEM((8 * 1024,), jnp.float32) for _ in range(2)]
            + [pltpu.SemaphoreType.DMA for _ in range(4)]
        ),
        compiler_params=pltpu.CompilerParams(
            use_tc_tiling_on_sc=False, needs_layout_passes=False
        ),
    )
    def k(x_hbm, tr_hbm, o_hbm, idx_v, g0, g1, ob0, ob1, sg0, sg1, so0, so1):
        wid = lax.axis_index("s") * _NC + lax.axis_index("c")
        iotas = [lax.iota(jnp.int32, _L) + (q * _L) for q in range(8)]
        gs, obs = (g0, g1), (ob0, ob1)
        sgs, sos = (sg0, sg1), (so0, so1)
        pltpu.sync_copy(x_hbm.at[:, pl.ds(wid * 128, 128)], idx_v)

        def out_off(s, db):
            return s * (D * B) + (db * 32 + wid) * 1024

        def fire_g(s, b):
            pltpu.async_copy(tr_hbm.at[idx_v.at[s]], gs[b], sgs[b])

        def wait_g(s, b):
            pltpu.make_async_copy(tr_hbm.at[idx_v.at[s]], gs[b], sgs[b]).wait()

        def fire_o(s, b):
            for db in range(8):
                pltpu.async_copy(
                    obs[b].at[pl.ds(db * 1024, 1024)],
                    o_hbm.at[pl.ds(out_off(s, db), 1024)],
                    sos[b],
                )

        def wait_o(s, b):
            for db in range(8):
                pltpu.make_async_copy(
                    obs[b].at[pl.ds(db * 1024, 1024)],
                    o_hbm.at[pl.ds(out_off(s, db), 1024)],
                    sos[b],
                ).wait()

        def chunk(s, b):
            wait_g(s, b)
            g, ob = gs[b], obs[b]

            @plsc.parallel_loop(0, D, unroll=2)
            def _(d):
                col = jnp.full((_L,), 1, jnp.int32) * d
                base = lax.shift_right_logical(d, 3) * 1024 + lax.rem(d, 8) * 128
                for q in range(8):
                    vals = plsc.load_gather(g, [iotas[q], col])
                    ob[pl.ds(base + q * _L, _L)] = vals * _SCALE

        fire_g(0, 0)
        fire_g(1, 1)

        def pair_body(s2, carry):
            for b in range(2):
                s = s2 * 2 + b

                @pl.when(s >= 2)
                def _():
                    wait_o(s - 2, b)

                chunk(s, b)
                fire_o(s, b)

                @pl.when(s + 2 < S)
                def _():
                    fire_g(s + 2, b)
            return carry

        lax.fori_loop(0, S // 2, pair_body, 0)
        wait_o(S - 2, 0)
        wait_o(S - 1, 1)

    return k(x2, t_row)


def kernel(x, table):
    S, B = x.shape[1], x.shape[0]
    tableT = table.T
    tail = table[_VMAIN:, :]
    t_row = _sc_rowify(tableT, tail)
    x2 = x.T.astype(jnp.int32)
    o = _sc_lookup(x2, t_row)
    return (
        o.reshape(S, 8, 32, 8, 128)
        .transpose(2, 4, 0, 1, 3)
        .reshape(B, S, D)
    )


# speedup vs baseline: 4.2666x; 4.2666x over previous
import functools

import jax
import jax.numpy as jnp
from jax import lax
from jax.experimental import pallas as pl
from jax.experimental.pallas import tpu as pltpu
from jax.experimental.pallas import tpu_sc as plsc

D = 64
_SCALE = 8.0
_NC, _NS, _L = 2, 16, 16
V = 1_000_000


@jax.jit
def _sc_lookup(x2, t_row):
    S, B = x2.shape
    mesh = plsc.VectorSubcoreMesh(core_axis_name="c", subcore_axis_name="s")

    @functools.partial(
        pl.kernel,
        mesh=mesh,
        out_type=jax.ShapeDtypeStruct((S * D * B,), jnp.float32),
        scratch_types=(
            [pltpu.VMEM((S, 128), jnp.int32)]
            + [pltpu.VMEM((128, D), jnp.float32) for _ in range(2)]
            + [pltpu.VMEM((8 * 1024,), jnp.float32) for _ in range(2)]
            + [pltpu.SemaphoreType.DMA for _ in range(4)]
        ),
        compiler_params=pltpu.CompilerParams(
            use_tc_tiling_on_sc=False, needs_layout_passes=False
        ),
    )
    def k(x_hbm, tr_hbm, o_hbm, idx_v, g0, g1, ob0, ob1, sg0, sg1, so0, so1):
        wid = lax.axis_index("s") * _NC + lax.axis_index("c")
        iotas = [lax.iota(jnp.int32, _L) + (q * _L) for q in range(8)]
        gs, obs = (g0, g1), (ob0, ob1)
        sgs, sos = (sg0, sg1), (so0, so1)
        pltpu.sync_copy(x_hbm.at[:, pl.ds(wid * 128, 128)], idx_v)

        def out_off(s, db):
            return s * (D * B) + (db * 32 + wid) * 1024

        def fire_g(s, b):
            pltpu.async_copy(tr_hbm.at[idx_v.at[s]], gs[b], sgs[b])

        def wait_g(s, b):
            pltpu.make_async_copy(tr_hbm.at[idx_v.at[s]], gs[b], sgs[b]).wait()

        def fire_o(s, b):
            for db in range(8):
                pltpu.async_copy(
                    obs[b].at[pl.ds(db * 1024, 1024)],
                    o_hbm.at[pl.ds(out_off(s, db), 1024)],
                    sos[b],
                )

        def wait_o(s, b):
            for db in range(8):
                pltpu.make_async_copy(
                    obs[b].at[pl.ds(db * 1024, 1024)],
                    o_hbm.at[pl.ds(out_off(s, db), 1024)],
                    sos[b],
                ).wait()

        def chunk(s, b):
            wait_g(s, b)
            g, ob = gs[b], obs[b]

            @plsc.parallel_loop(0, 8, unroll=2)
            def _(db):
                col0 = jnp.full((_L,), 1, jnp.int32) * (db * 8)
                base = db * 1024
                for din in range(8):
                    col = col0 + din
                    for q in range(8):
                        vals = plsc.load_gather(g, [iotas[q], col])
                        ob[pl.ds(base + din * 128 + q * _L, _L)] = vals * _SCALE

        fire_g(0, 0)
        fire_g(1, 1)

        def pair_body(s2, carry):
            for b in range(2):
                s = s2 * 2 + b

                @pl.when(s >= 2)
                def _():
                    wait_o(s - 2, b)

                chunk(s, b)
                fire_o(s, b)

                @pl.when(s + 2 < S)
                def _():
                    fire_g(s + 2, b)
            return carry

        lax.fori_loop(0, S // 2, pair_body, 0)
        wait_o(S - 2, 0)
        wait_o(S - 1, 1)

    return k(x2, t_row)


def kernel(x, table):
    S, B = x.shape[1], x.shape[0]
    x2 = x.T.astype(jnp.int32)
    o = _sc_lookup(x2, table)
    return (
        o.reshape(S, 8, 32, 8, 128)
        .transpose(2, 4, 0, 1, 3)
        .reshape(B, S, D)
    )


# speedup vs baseline: 5.2280x; 1.2254x over previous
import functools

import jax
import jax.numpy as jnp
from jax import lax
from jax.experimental import pallas as pl
from jax.experimental.pallas import tpu as pltpu
from jax.experimental.pallas import tpu_sc as plsc

D_MODEL = 64
_SCALE = 8.0
_NC, _NS, _L = 2, 16, 16
_NW = _NC * _NS
_C = 256
_NBUF = 4


@functools.partial(jax.jit, static_argnums=(0,))
def _sc_lookup(B, xf, table):
    per_w = B // _NW
    n_chunks = per_w // _C
    groups = n_chunks // _NBUF
    mesh = plsc.VectorSubcoreMesh(core_axis_name="c", subcore_axis_name="s")

    @functools.partial(
        pl.kernel,
        mesh=mesh,
        out_type=jax.ShapeDtypeStruct((B, D_MODEL), jnp.float32),
        scratch_types=(
            [pltpu.VMEM((per_w,), jnp.int32)]
            + [pltpu.VMEM((_C, D_MODEL), jnp.float32) for _ in range(_NBUF)]
            + [pltpu.SemaphoreType.DMA for _ in range(2 * _NBUF)]
        ),
        compiler_params=pltpu.CompilerParams(use_tc_tiling_on_sc=False),
    )
    def k(x_hbm, table_hbm, out_hbm, idx_v, *bufs_and_sems):
        rows = bufs_and_sems[:_NBUF]
        gsem = bufs_and_sems[_NBUF : 2 * _NBUF]
        ssem = bufs_and_sems[2 * _NBUF :]
        wid = lax.axis_index("s") * _NC + lax.axis_index("c")
        base = wid * per_w
        pltpu.sync_copy(x_hbm.at[pl.ds(base, per_w)], idx_v)

        def fire_g(c, b):
            pltpu.async_copy(
                table_hbm.at[idx_v.at[pl.ds(c * _C, _C)]], rows[b], gsem[b]
            )

        def wait_g(c, b):
            pltpu.make_async_copy(
                table_hbm.at[idx_v.at[pl.ds(c * _C, _C)]], rows[b], gsem[b]
            ).wait()

        def fire_s(c, b):
            pltpu.async_copy(rows[b], out_hbm.at[pl.ds(base + c * _C, _C)], ssem[b])

        def wait_s(c, b):
            pltpu.make_async_copy(
                rows[b], out_hbm.at[pl.ds(base + c * _C, _C)], ssem[b]
            ).wait()

        def scale(b):
            buf = rows[b]

            @plsc.parallel_loop(0, _C, unroll=4)
            def _(i):
                for j in range(D_MODEL // _L):
                    s = (i, pl.ds(j * _L, _L))
                    buf[s] = buf[s] * _SCALE

        def chunk_body(c, b, refire):
            wait_g(c, b)
            scale(b)
            fire_s(c, b)
            if refire:
                bp = (b - 1) % _NBUF
                wait_s(c - 1, bp)
                fire_g(c - 1 + _NBUF, bp)

        for b in range(_NBUF):
            fire_g(b, b)
        for b in range(_NBUF):
            chunk_body(b, b, refire=b >= 1)

        def group_body(g, carry):
            c0 = g * _NBUF
            for b in range(_NBUF):
                chunk_body(c0 + b, b, refire=True)
            return carry

        lax.fori_loop(1, groups - 1, group_body, 0)

        c0 = (groups - 1) * _NBUF
        for b in range(_NBUF):
            chunk_body(c0 + b, b, refire=b == 0)
        for b in range(_NBUF):
            wait_s(c0 + b, b)

    return k(xf, table)


def kernel(x, table):
    lead_shape = x.shape
    xf = x.reshape(-1).astype(jnp.int32)
    out = _sc_lookup(xf.shape[0], xf, table)
    return out.reshape(*lead_shape, D_MODEL)
